# Initial kernel scaffold; baseline (speedup 1.0000x reference)
import jax
import jax.numpy as jnp
from jax.experimental import pallas as pl


def kernel(z_e_x, codebook):
    raise NotImplementedError("write your pallas kernel here")


# speedup vs baseline: 1.3757x; 1.3757x over previous
import jax
import jax.numpy as jnp
from jax.experimental import pallas as pl


_BM = 512


def _vq_block(x_ref, c_ref, csqr_ref, isqr_ref, o_ref):
    x = x_ref[...].astype(jnp.bfloat16)
    c = c_ref[...].astype(jnp.bfloat16)
    s = jax.lax.dot_general(
        x, c, (((1,), (1,)), ((), ())), preferred_element_type=jnp.float32
    )
    d = (csqr_ref[...] + isqr_ref[...]) - 2.0 * s
    idx = jnp.argmin(d, axis=1).astype(jnp.int32)
    o_ref[0, 0, :] = idx


def kernel(z_e_x, codebook):
    K, D = codebook.shape
    x = z_e_x.reshape(-1, D)
    M = x.shape[0]
    nm = M // _BM

    c_sqr = jnp.sum(codebook ** 2, axis=1).reshape(1, K)
    i_sqr = jnp.sum(x ** 2, axis=1, keepdims=True)

    out = pl.pallas_call(
        _vq_block,
        grid=(nm,),
        in_specs=[
            pl.BlockSpec((_BM, D), lambda i: (i, 0)),
            pl.BlockSpec((K, D), lambda i: (0, 0)),
            pl.BlockSpec((1, K), lambda i: (0, 0)),
            pl.BlockSpec((_BM, 1), lambda i: (i, 0)),
        ],
        out_specs=pl.BlockSpec((1, 1, _BM), lambda i: (i, 0, 0)),
        out_shape=jax.ShapeDtypeStruct((nm, 1, _BM), jnp.int32),
    )(x, codebook, c_sqr, i_sqr)

    return out.reshape(z_e_x.shape[:-1])
